# baseline (speedup 1.0000x reference)
import functools

import jax
import jax.numpy as jnp
from jax import lax
from jax.experimental import pallas as pl
from jax.experimental.pallas import tpu as pltpu
from jax.experimental.pallas import tpu_sc as plsc

_BATCH = 4096
_MAX_LEN = 200
_EMBED = 64
_VOCAB = 32

_LANES = 16
_BC = 128
_NBCH = _BATCH // _BC
_NTG = 8
_TPG = _MAX_LEN // _NTG
_FPITCH = _EMBED + 1
_FROWS = _VOCAB * _FPITCH


def _prep_body(seq_ref, tab_ref, pos_ref, seqt_ref, fus_ref):
    for c in range(_NBCH):
        blk = seq_ref[pl.ds(c * _BC, _BC), :]
        seqt_ref[:, c, :] = blk.T
    tab = tab_ref[...]
    pos = pos_ref[...]
    fus_ref[:, :, 0:_EMBED] = pos[:, None, :] + tab[None, :, :]


def _prep(seq, seq_table, pos_table):
    return pl.pallas_call(
        _prep_body,
        out_shape=(
            jax.ShapeDtypeStruct((_MAX_LEN, _NBCH, _BC), jnp.int32),
            jax.ShapeDtypeStruct((_MAX_LEN, _VOCAB, _FPITCH), jnp.float32),
        ),
    )(seq, seq_table, pos_table)


def _sc_body(nc, nbg, seqt, fus, out, seq_v, fus_v, blk_v, ss0, ss1):
    sem_s = (ss0, ss1)
    wid = lax.axis_index("s") * nc + lax.axis_index("c")
    tg = wid // nbg
    bg = wid % nbg
    t0 = tg * _TPG
    bch0 = bg * (_NBCH // nbg)

    pltpu.sync_copy(fus.at[pl.ds(t0 * _FROWS, _TPG * _FROWS)], fus_v)
    pltpu.sync_copy(seqt.at[pl.ds(t0, _TPG), pl.ds(bch0, _NBCH // nbg)],
                    seq_v)

    n_chunks = _TPG * (_NBCH // nbg)

    def do_chunk(chunk, buf):
        tl = chunk // (_NBCH // nbg)
        bc = chunk % (_NBCH // nbg)

        @pl.when(chunk >= 2)
        def _():
            pltpu.make_async_copy(
                blk_v.at[buf, :, pl.ds(0, _BC)],
                out.at[pl.ds(0, _EMBED), pl.ds(0, _BC)],
                sem_s[buf]).wait()

        iota = lax.iota(jnp.int32, _LANES)

        def jbody(j0, c):
            seq16 = seq_v[tl, bc, pl.ds(j0 * _LANES, _LANES)]
            base16 = seq16 * _FPITCH + tl * _FROWS
            for jj in range(_LANES):
                bb = jnp.broadcast_to(base16[jj], (_LANES,))
                jvec = jnp.full((_LANES,), j0 * _LANES + jj, jnp.int32)
                for k in range(_EMBED // _LANES):
                    evec = iota + k * _LANES
                    val = plsc.load_gather(fus_v, [bb + evec])
                    plsc.store_scatter(blk_v.at[buf], [evec, jvec], val)
            return c

        lax.fori_loop(0, _BC // _LANES, jbody, 0)

        t = t0 + tl
        b0 = bch0 * _BC + bc * _BC
        pltpu.async_copy(
            blk_v.at[buf, :, pl.ds(0, _BC)],
            out.at[pl.ds(t * _EMBED, _EMBED), pl.ds(b0, _BC)],
            sem_s[buf])

    def body(i, carry):
        for buf in range(2):
            do_chunk(i * 2 + buf, buf)
        return carry

    lax.fori_loop(0, n_chunks // 2, body, 0)
    for buf in range(2):
        pltpu.make_async_copy(
            blk_v.at[buf, :, pl.ds(0, _BC)],
            out.at[pl.ds(0, _EMBED), pl.ds(0, _BC)],
            sem_s[buf]).wait()


def _sc_gather(seqt, fus_flat):
    info = plsc.get_sparse_core_info()
    nc, ns = info.num_cores, info.num_subcores
    nbg = (nc * ns) // _NTG
    mesh = plsc.VectorSubcoreMesh(core_axis_name="c", subcore_axis_name="s")
    kern = pl.kernel(
        functools.partial(_sc_body, nc, nbg),
        out_type=jax.ShapeDtypeStruct((_MAX_LEN * _EMBED, _BATCH),
                                      jnp.float32),
        mesh=mesh,
        compiler_params=pltpu.CompilerParams(use_tc_tiling_on_sc=True,
                                             needs_layout_passes=False),
        scratch_types=[
            pltpu.VMEM((_TPG, _NBCH // nbg, _BC), jnp.int32),
            pltpu.VMEM((_TPG * _FROWS,), jnp.float32),
            pltpu.VMEM((2, _EMBED, _BC + 1), jnp.float32),
            pltpu.SemaphoreType.DMA,
            pltpu.SemaphoreType.DMA,
        ],
    )
    return kern(seqt, fus_flat)


def kernel(seq, seq_table, pos_table):
    seqt, fus2 = _prep(seq.astype(jnp.int32),
                       seq_table.astype(jnp.float32),
                       pos_table.astype(jnp.float32))
    fus_flat = fus2.reshape(_MAX_LEN * _FROWS)
    out2 = _sc_gather(seqt, fus_flat)
    out3 = out2.reshape(_MAX_LEN, _EMBED, _BATCH)
    return jnp.transpose(out3, (2, 0, 1))


# speedup vs baseline: 2.6941x; 2.6941x over previous
import functools

import jax
import jax.numpy as jnp
from jax import lax
from jax.experimental import pallas as pl
from jax.experimental.pallas import tpu as pltpu
from jax.experimental.pallas import tpu_sc as plsc

_BATCH = 4096
_MAX_LEN = 200
_EMBED = 64
_VOCAB = 32

_LANES = 16
_BC = 128
_NBCH = _BATCH // _BC
_NTG = 8
_TPG = _MAX_LEN // _NTG
_FPITCH = _EMBED + 1
_FROWS = _VOCAB * _FPITCH


def _prep_body(seq_ref, tab_ref, pos_ref, seqt_ref, fus_ref):
    for c in range(_NBCH):
        blk = seq_ref[pl.ds(c * _BC, _BC), :]
        seqt_ref[:, c, :] = blk.T
    tab = tab_ref[...]
    pos = pos_ref[...]
    fus_ref[:, :, 0:_EMBED] = pos[:, None, :] + tab[None, :, :]


def _prep(seq, seq_table, pos_table):
    return pl.pallas_call(
        _prep_body,
        out_shape=(
            jax.ShapeDtypeStruct((_MAX_LEN, _NBCH, _BC), jnp.int32),
            jax.ShapeDtypeStruct((_MAX_LEN, _VOCAB, _FPITCH), jnp.float32),
        ),
    )(seq, seq_table, pos_table)


def _sc_body(nc, nbg, seqt, fus, out, seq_v, fus_v, blk_v, ss0, ss1):
    sem_s = (ss0, ss1)
    wid = lax.axis_index("s") * nc + lax.axis_index("c")
    tg = wid // nbg
    bg = wid % nbg
    t0 = tg * _TPG
    bch0 = bg * (_NBCH // nbg)

    pltpu.sync_copy(fus.at[pl.ds(t0 * _FROWS, _TPG * _FROWS)], fus_v)
    pltpu.sync_copy(seqt.at[pl.ds(t0, _TPG), pl.ds(bch0, _NBCH // nbg)],
                    seq_v)

    n_chunks = _TPG * (_NBCH // nbg)

    def do_chunk(chunk, buf):
        tl = chunk // (_NBCH // nbg)
        bc = chunk % (_NBCH // nbg)

        @pl.when(chunk >= 2)
        def _():
            pltpu.make_async_copy(
                blk_v.at[buf],
                out.at[pl.ds(0, _EMBED), pl.ds(0, _BC)],
                sem_s[buf]).wait()

        for j0 in range(_BC // _LANES):
            seq16 = seq_v[tl, bc, pl.ds(j0 * _LANES, _LANES)]
            addr0 = seq16 * _FPITCH + tl * _FROWS
            sl = pl.ds(j0 * _LANES, _LANES)
            for e0 in range(0, _EMBED, 4):
                vals = [plsc.load_gather(fus_v, [addr0 + (e0 + d)])
                        for d in range(4)]
                for d in range(4):
                    blk_v[buf, e0 + d, sl] = vals[d]

        t = t0 + tl
        b0 = bch0 * _BC + bc * _BC
        pltpu.async_copy(
            blk_v.at[buf],
            out.at[pl.ds(t * _EMBED, _EMBED), pl.ds(b0, _BC)],
            sem_s[buf])

    def body(i, carry):
        for buf in range(2):
            do_chunk(i * 2 + buf, buf)
        return carry

    lax.fori_loop(0, n_chunks // 2, body, 0)
    for buf in range(2):
        pltpu.make_async_copy(
            blk_v.at[buf],
            out.at[pl.ds(0, _EMBED), pl.ds(0, _BC)],
            sem_s[buf]).wait()


def _sc_gather(seqt, fus_flat):
    info = plsc.get_sparse_core_info()
    nc, ns = info.num_cores, info.num_subcores
    nbg = (nc * ns) // _NTG
    mesh = plsc.VectorSubcoreMesh(core_axis_name="c", subcore_axis_name="s")
    kern = pl.kernel(
        functools.partial(_sc_body, nc, nbg),
        out_type=jax.ShapeDtypeStruct((_MAX_LEN * _EMBED, _BATCH),
                                      jnp.float32),
        mesh=mesh,
        compiler_params=pltpu.CompilerParams(use_tc_tiling_on_sc=True,
                                             needs_layout_passes=False),
        scratch_types=[
            pltpu.VMEM((_TPG, _NBCH // nbg, _BC), jnp.int32),
            pltpu.VMEM((_TPG * _FROWS,), jnp.float32),
            pltpu.VMEM((2, _EMBED, _BC), jnp.float32),
            pltpu.SemaphoreType.DMA,
            pltpu.SemaphoreType.DMA,
        ],
    )
    return kern(seqt, fus_flat)


def kernel(seq, seq_table, pos_table):
    seqt, fus2 = _prep(seq.astype(jnp.int32),
                       seq_table.astype(jnp.float32),
                       pos_table.astype(jnp.float32))
    fus_flat = fus2.reshape(_MAX_LEN * _FROWS)
    out2 = _sc_gather(seqt, fus_flat)
    out3 = out2.reshape(_MAX_LEN, _EMBED, _BATCH)
    return jnp.transpose(out3, (2, 0, 1))


# speedup vs baseline: 3.2000x; 1.1878x over previous
import functools

import jax
import jax.numpy as jnp
from jax import lax
from jax.experimental import pallas as pl
from jax.experimental.pallas import tpu as pltpu
from jax.experimental.pallas import tpu_sc as plsc

_BATCH = 4096
_MAX_LEN = 200
_EMBED = 64
_VOCAB = 32

_LANES = 16
_BC = 128
_NBCH = _BATCH // _BC
_NTG = 8
_TPG = _MAX_LEN // _NTG
_FPITCH = _EMBED + 1
_FROWS = _VOCAB * _FPITCH


def _prep_body(seq_ref, tab_ref, pos_ref, seqt_ref, fus_ref):
    for c in range(_NBCH):
        blk = seq_ref[pl.ds(c * _BC, _BC), :]
        seqt_ref[:, c, :] = blk.T
    tab = tab_ref[...]
    pos = pos_ref[...]
    fus_ref[:, :, 0:_EMBED] = pos[:, None, :] + tab[None, :, :]


def _prep(seq, seq_table, pos_table):
    return pl.pallas_call(
        _prep_body,
        out_shape=(
            jax.ShapeDtypeStruct((_MAX_LEN, _NBCH, _BC), jnp.int32),
            jax.ShapeDtypeStruct((_MAX_LEN, _VOCAB, _FPITCH), jnp.float32),
        ),
    )(seq, seq_table, pos_table)


def _sc_body(nc, nbg, seqt, fus, out, seq_v, fus_v, blk_v, ss0, ss1):
    sem_s = (ss0, ss1)
    wid = lax.axis_index("s") * nc + lax.axis_index("c")
    tg = wid // nbg
    bg = wid % nbg
    t0 = tg * _TPG
    bch0 = bg * (_NBCH // nbg)

    pltpu.sync_copy(fus.at[pl.ds(t0 * _FROWS, _TPG * _FROWS)], fus_v)
    pltpu.sync_copy(seqt.at[pl.ds(t0, _TPG), pl.ds(bch0, _NBCH // nbg)],
                    seq_v)

    n_chunks = _TPG * (_NBCH // nbg)

    def do_chunk(chunk, buf):
        tl = chunk // (_NBCH // nbg)
        bc = chunk % (_NBCH // nbg)

        @pl.when(chunk >= 2)
        def _():
            pltpu.make_async_copy(
                blk_v.at[buf],
                out.at[pl.ds(0, _EMBED), pl.ds(0, _BC)],
                sem_s[buf]).wait()

        for j0 in range(_BC // _LANES):
            seq16 = seq_v[tl, bc, pl.ds(j0 * _LANES, _LANES)]
            addr0 = seq16 * _FPITCH + tl * _FROWS
            sl = pl.ds(j0 * _LANES, _LANES)
            for e0 in range(0, _EMBED, 8):
                vals = [plsc.load_gather(fus_v, [addr0 + (e0 + d)])
                        for d in range(8)]
                for d in range(8):
                    blk_v[buf, e0 + d, sl] = vals[d]

        t = t0 + tl
        b0 = bch0 * _BC + bc * _BC
        pltpu.async_copy(
            blk_v.at[buf],
            out.at[pl.ds(t * _EMBED, _EMBED), pl.ds(b0, _BC)],
            sem_s[buf])

    def body(i, carry):
        for buf in range(2):
            do_chunk(i * 2 + buf, buf)
        return carry

    lax.fori_loop(0, n_chunks // 2, body, 0)
    for buf in range(2):
        pltpu.make_async_copy(
            blk_v.at[buf],
            out.at[pl.ds(0, _EMBED), pl.ds(0, _BC)],
            sem_s[buf]).wait()


def _sc_gather(seqt, fus_flat):
    info = plsc.get_sparse_core_info()
    nc, ns = info.num_cores, info.num_subcores
    nbg = (nc * ns) // _NTG
    mesh = plsc.VectorSubcoreMesh(core_axis_name="c", subcore_axis_name="s")
    kern = pl.kernel(
        functools.partial(_sc_body, nc, nbg),
        out_type=jax.ShapeDtypeStruct((_MAX_LEN * _EMBED, _BATCH),
                                      jnp.float32),
        mesh=mesh,
        compiler_params=pltpu.CompilerParams(use_tc_tiling_on_sc=True,
                                             needs_layout_passes=False),
        scratch_types=[
            pltpu.VMEM((_TPG, _NBCH // nbg, _BC), jnp.int32),
            pltpu.VMEM((_TPG * _FROWS,), jnp.float32),
            pltpu.VMEM((2, _EMBED, _BC), jnp.float32),
            pltpu.SemaphoreType.DMA,
            pltpu.SemaphoreType.DMA,
        ],
    )
    return kern(seqt, fus_flat)


def kernel(seq, seq_table, pos_table):
    seqt, fus2 = _prep(seq.astype(jnp.int32),
                       seq_table.astype(jnp.float32),
                       pos_table.astype(jnp.float32))
    fus_flat = fus2.reshape(_MAX_LEN * _FROWS)
    out2 = _sc_gather(seqt, fus_flat)
    out3 = out2.reshape(_MAX_LEN, _EMBED, _BATCH)
    return jnp.transpose(out3, (2, 0, 1))
